# baseline (speedup 1.0000x reference)
import functools

import jax
import jax.numpy as jnp
from jax import lax
from jax.experimental import pallas as pl
from jax.experimental.pallas import tpu as pltpu
from jax.experimental.pallas import tpu_sc as plsc

H = 384
HW = H * H
ACT = 192
PAD = ACT + 2
RSTR = PAD * 3
CANW = PAD * PAD * 3 + 4
BW = 37760
BSL = BW // 8
CHUNK = 1152
NCH = HW // CHUNK
PPT = HW // 8
ROWW = 8 * H
BROWW = 8 * PAD
EPS = 1e-10

_mesh = plsc.VectorSubcoreMesh(core_axis_name="c", subcore_axis_name="s")


@functools.partial(
    pl.kernel,
    mesh=_mesh,
    compiler_params=pltpu.CompilerParams(needs_layout_passes=False),
    out_type=(
        jax.ShapeDtypeStruct((2, 96, HW), jnp.float32),
        jax.ShapeDtypeStruct((32 * BW,), jnp.float32),
        jax.ShapeDtypeStruct((2 * BW,), jnp.float32),
    ),
    scratch_types=[
        pltpu.VMEM((CANW,), jnp.float32),
        [pltpu.VMEM((CHUNK,), jnp.float32) for _ in range(5)],
        [pltpu.VMEM((CHUNK,), jnp.float32) for _ in range(5)],
        pltpu.VMEM((ROWW,), jnp.float32),
        pltpu.VMEM((1568,), jnp.float32),
        pltpu.SemaphoreType.DMA,
        pltpu.SemaphoreType.DMA,
    ],
)
def _splat_sc(x_hbm, gx_hbm, gy_hbm, out_hbm, bpart_hbm, bfin_hbm,
              canvas, bufs_a, bufs_b, rowbuf, bstage, sem_a, sem_b):
    core = lax.axis_index("c")
    sub = lax.axis_index("s")
    wid = sub * 2 + core
    iota = lax.iota(jnp.int32, 16)
    zeros16 = jnp.zeros((16,), jnp.float32)
    ones16 = jnp.ones((16,), jnp.float32)

    def fill(ref, nwords, val):
        def body(i, _):
            ref[pl.ds(i * 16, 16)] = val
            return 0
        lax.fori_loop(0, nwords // 16, body, 0)

    def coords16(gxb, gyb, o):
        gx = gxb[pl.ds(o, 16)] * 192.0 + 193.0
        gy = gyb[pl.ds(o, 16)] * 192.0 + 193.0
        ix = gx.astype(jnp.int32)
        iy = gy.astype(jnp.int32)
        tx = gx - ix.astype(jnp.float32)
        ty = gy - iy.astype(jnp.float32)
        wr0 = jnp.clip(ix - 193, 0, ACT)
        wc0 = jnp.clip(iy - 193, 0, ACT)
        return wr0, wc0, tx, ty

    fill(rowbuf, ROWW, ones16)

    bb = sub // 8
    part = sub % 8
    fill(canvas, BW, zeros16)
    gxa, gya = bufs_a[0], bufs_a[1]

    def b_chunk(k, _):
        off = part * PPT + k * CHUNK
        pltpu.sync_copy(gx_hbm.at[bb, pl.ds(off, CHUNK)], gxa)
        pltpu.sync_copy(gy_hbm.at[bb, pl.ds(off, CHUNK)], gya)

        @plsc.parallel_loop(0, CHUNK // 16, 1, unroll=4)
        def px_body(j):
            o = j * 16
            wr0, wc0, tx, ty = coords16(gxa, gya, o)
            ux = 1.0 - tx
            uy = 1.0 - ty
            a00 = wr0 * PAD + wc0
            plsc.addupdate_scatter(canvas, [a00], ux * uy)
            plsc.addupdate_scatter(canvas, [a00 + 1], ux * ty)
            plsc.addupdate_scatter(canvas, [a00 + PAD], tx * uy)
            plsc.addupdate_scatter(canvas, [a00 + PAD + 1], tx * ty)
        return 0

    lax.fori_loop(0, PPT // CHUNK, b_chunk, 0)
    prow = (core * 2 + bb) * 8 + part
    pltpu.sync_copy(canvas.at[pl.ds(0, BW)], bpart_hbm.at[pl.ds(prow * BW, BW)])
    plsc.subcore_barrier()

    acc0 = 40960
    tmp0 = 49152
    pltpu.sync_copy(bpart_hbm.at[pl.ds((core * 2 + bb) * 8 * BW + part * BSL, BSL)],
                    canvas.at[pl.ds(acc0, BSL)])
    for t in range(1, 8):
        pltpu.sync_copy(bpart_hbm.at[pl.ds(((core * 2 + bb) * 8 + t) * BW + part * BSL, BSL)],
                        canvas.at[pl.ds(tmp0, BSL)])

        def addb(i, _):
            o1 = acc0 + i * 16
            o2 = tmp0 + i * 16
            canvas[pl.ds(o1, 16)] = canvas[pl.ds(o1, 16)] + canvas[pl.ds(o2, 16)]
            return 0
        lax.fori_loop(0, BSL // 16, addb, 0)
    pltpu.sync_copy(canvas.at[pl.ds(acc0, BSL)],
                    bfin_hbm.at[pl.ds(bb * BW + part * BSL, BSL)])
    plsc.subcore_barrier()

    for p in range(2):
        t3 = p * 32 + wid
        plane0 = t3 * 3
        b = plane0 // 96
        c0 = plane0 % 96

        fill(canvas, CANW, zeros16)

        def srcs(k):
            off = k * CHUNK
            return (gx_hbm.at[b, pl.ds(off, CHUNK)],
                    gy_hbm.at[b, pl.ds(off, CHUNK)],
                    x_hbm.at[b, c0, pl.ds(off, CHUNK)],
                    x_hbm.at[b, c0 + 1, pl.ds(off, CHUNK)],
                    x_hbm.at[b, c0 + 2, pl.ds(off, CHUNK)])

        def issue(bufs, sem, k):
            for s, d in zip(srcs(k), bufs):
                pltpu.async_copy(s, d, sem)

        def drain(bufs, sem):
            for s, d in zip(srcs(0), bufs):
                pltpu.make_async_copy(s, d, sem).wait()

        def compute(bufs):
            gxb, gyb, xb0, xb1, xb2 = bufs

            def splat16(o):
                vx0 = xb0[pl.ds(o, 16)]
                vx1 = xb1[pl.ds(o, 16)]
                vx2 = xb2[pl.ds(o, 16)]
                wr0, wc0, tx, ty = coords16(gxb, gyb, o)
                ux = 1.0 - tx
                uy = 1.0 - ty
                a00 = wr0 * RSTR + wc0 * 3
                w00 = ux * uy
                w01 = ux * ty
                w10 = tx * uy
                w11 = tx * ty
                for (di, dj, w) in ((0, 0, w00), (0, 1, w01),
                                    (1, 0, w10), (1, 1, w11)):
                    addr = a00 + (di * RSTR + dj * 3)
                    plsc.addupdate_scatter(canvas, [addr], w * vx0)
                    plsc.addupdate_scatter(canvas, [addr + 1], w * vx1)
                    plsc.addupdate_scatter(canvas, [addr + 2], w * vx2)

            @plsc.parallel_loop(0, CHUNK // 16, 1, unroll=8)
            def px_body(j):
                splat16(j * 16)

        issue(bufs_a, sem_a, 0)

        def chunk2_body(k, _):
            issue(bufs_b, sem_b, 2 * k + 1)
            drain(bufs_a, sem_a)
            compute(bufs_a)
            issue(bufs_a, sem_a, jnp.minimum(2 * k + 2, NCH - 2))
            drain(bufs_b, sem_b)
            compute(bufs_b)
            return 0

        lax.fori_loop(0, NCH // 2, chunk2_body, 0)
        drain(bufs_a, sem_a)

        def right_ones(ir, _):
            def col_ones(v, _):
                rowbuf[pl.ds(ir * H + ACT + v * 16, 16)] = ones16
                return 0
            lax.fori_loop(0, ACT // 16, col_ones, 0)
            return 0
        lax.fori_loop(0, 8, right_ones, 0)

        for ch in range(3):
            def top_body(rc, _):
                pltpu.sync_copy(
                    rowbuf, out_hbm.at[b, c0 + ch, pl.ds(rc * ROWW, ROWW)])
                return 0
            lax.fori_loop(0, (ACT * H) // ROWW, top_body, 0)

        def bot_body(rc, _):
            pltpu.sync_copy(bfin_hbm.at[pl.ds(b * BW + rc * BROWW, BROWW)],
                            bstage.at[pl.ds(0, BROWW)])
            for ch in range(3):
                def row_body(ir, _):
                    base3 = (rc * 8 + ir) * RSTR
                    baseb = ir * PAD

                    def finish16(v):
                        a3 = base3 + (v * 16 + iota) * 3
                        av = plsc.load_gather(canvas, [a3 + ch])
                        bv = bstage[pl.ds(baseb + v * 16, 16)]
                        outv = jnp.where(bv > EPS, av / (bv + EPS), 1.0)
                        rowbuf[pl.ds(ir * H + ACT + v * 16, 16)] = outv

                    plsc.parallel_loop(0, ACT // 16, 1, unroll=4)(finish16)
                    return 0
                lax.fori_loop(0, 8, row_body, 0)
                pltpu.sync_copy(
                    rowbuf,
                    out_hbm.at[b, c0 + ch, pl.ds(ACT * H + rc * ROWW, ROWW)])
            return 0
        lax.fori_loop(0, (ACT * H) // ROWW, bot_body, 0)


def kernel(x, inv_grid):
    b, c, h, w = x.shape
    xf = x.reshape(b, c, h * w)
    gxp = inv_grid[..., 0].reshape(b, h * w)
    gyp = inv_grid[..., 1].reshape(b, h * w)
    out, _, _ = _splat_sc(xf, gxp, gyp)
    return out.reshape(b, c, h, w)
